# Initial kernel scaffold; baseline (speedup 1.0000x reference)
import jax
import jax.numpy as jnp
from jax.experimental import pallas as pl


def kernel(pts_co, params):
    raise NotImplementedError("write your pallas kernel here")


# speedup vs baseline: 1.4019x; 1.4019x over previous
import functools

import jax
import jax.numpy as jnp
from jax.experimental import pallas as pl
from jax.experimental.pallas import tpu as pltpu


def _fps_body(M, pts_ref, samp_ref, cox_ref, coy_ref, coz_ref):
    xs = pts_ref[0]
    ys = pts_ref[1]
    zs = pts_ref[2]
    b, n = xs.shape
    lane = jax.lax.broadcasted_iota(jnp.int32, (b, n), 1)
    lane_m = jax.lax.broadcasted_iota(jnp.int32, (b, M), 1)
    big_i = jnp.int32(n)

    def step(i, carry):
        min_d, last, samples, sx, sy, sz = carry
        oh = lane == last
        lx = jnp.max(jnp.where(oh, xs, -jnp.inf), axis=1, keepdims=True)
        ly = jnp.max(jnp.where(oh, ys, -jnp.inf), axis=1, keepdims=True)
        lz = jnp.max(jnp.where(oh, zs, -jnp.inf), axis=1, keepdims=True)
        sel = lane_m == i
        samples = jnp.where(sel, jnp.broadcast_to(last, (b, M)), samples)
        sx = jnp.where(sel, jnp.broadcast_to(lx, (b, M)), sx)
        sy = jnp.where(sel, jnp.broadcast_to(ly, (b, M)), sy)
        sz = jnp.where(sel, jnp.broadcast_to(lz, (b, M)), sz)
        dx = xs - lx
        dy = ys - ly
        dz = zs - lz
        d = dx * dx + dy * dy + dz * dz
        min_d = jnp.minimum(min_d, d)
        m = jnp.max(min_d, axis=1, keepdims=True)
        nxt = jnp.min(jnp.where(min_d == m, lane, big_i), axis=1,
                      keepdims=True)
        return min_d, nxt, samples, sx, sy, sz

    row_m = jax.lax.broadcasted_iota(jnp.int32, (b, M), 0)
    varying = lane_m + row_m
    init = (
        jnp.full((b, n), 1e10, dtype=jnp.float32),
        jnp.zeros((b, 1), dtype=jnp.int32),
        varying,
        varying.astype(jnp.float32),
        varying.astype(jnp.float32),
        varying.astype(jnp.float32),
    )
    _, _, samples, sx, sy, sz = jax.lax.fori_loop(0, M, step, init)
    samp_ref[...] = samples
    cox_ref[...] = sx
    coy_ref[...] = sy
    coz_ref[...] = sz


def _fps(pts_co, M):
    b, n, _ = pts_co.shape
    pts_t = jnp.transpose(pts_co, (2, 0, 1))
    samp, sx, sy, sz = pl.pallas_call(
        functools.partial(_fps_body, M),
        out_shape=[
            jax.ShapeDtypeStruct((b, M), jnp.int32),
            jax.ShapeDtypeStruct((b, M), jnp.float32),
            jax.ShapeDtypeStruct((b, M), jnp.float32),
            jax.ShapeDtypeStruct((b, M), jnp.float32),
        ],
    )(pts_t)
    new_co = jnp.stack([sx, sy, sz], axis=-1)
    return samp, new_co


def _gather_nd(fea, idx):
    b = fea.shape[0]
    bi = jnp.arange(b).reshape((b,) + (1,) * (idx.ndim - 1))
    return fea[bi, idx]


def _bn_relu(x, g, beta):
    axes = tuple(range(x.ndim - 1))
    m = jnp.mean(x, axis=axes, keepdims=True)
    v = jnp.var(x, axis=axes, keepdims=True)
    return jax.nn.relu((x - m) / jnp.sqrt(v + 1e-5) * g + beta)


def _mlp(x, layers):
    for (W, bvec, g, beta) in layers:
        x = x @ W.T + bvec
        x = _bn_relu(x, g, beta)
    return x


def _sq_distance(a, b):
    return jnp.sum((a[:, :, None, :] - b[:, None, :, :]) ** 2, axis=-1)


def _sa_level(layers, pts_co, pts_fea, M, radius, G):
    _, new_co = _fps(pts_co, M)
    d2 = _sq_distance(new_co, pts_co)
    d = jnp.sqrt(d2 + 1e-12)
    order = jnp.argsort(d, axis=2)[:, :, :G]
    d_sorted = jnp.take_along_axis(d, order, axis=2)
    idx = jnp.where(d_sorted <= radius, order, order[:, :, :1])
    g_co = _gather_nd(pts_co, idx)
    rel = g_co - new_co[:, :, None, :]
    g_fea = _gather_nd(pts_fea, idx)
    x = jnp.concatenate([rel, g_fea], axis=3)
    x = _mlp(x, layers)
    new_fea = jnp.max(x, axis=2)
    return new_co, new_fea


def _global_feature(layers, pts_fea, n_sample):
    b, n, c = pts_fea.shape
    x = jnp.broadcast_to(pts_fea[:, None, :, :], (b, n_sample, n, c))
    x = _mlp(x, layers)
    return jnp.max(x, axis=2)[:, 0]


def _fp_interp(co_small, co_big, fea_big):
    d2 = _sq_distance(co_small, co_big)
    dist = jnp.sqrt(d2 + 1e-12)
    neighbors = jnp.argsort(dist, axis=2)
    sorted_dist = jnp.take_along_axis(dist, neighbors, axis=2)
    is_big = sorted_dist[:, :, 0] < 1e-5
    single = _gather_nd(fea_big, neighbors[:, :, 0])
    w = 1.0 / jnp.maximum(sorted_dist[:, :, :3], 1e-8)
    gathered = _gather_nd(fea_big, neighbors[:, :, :3])
    multiple = (jnp.sum(gathered * w[..., None], axis=2)
                / jnp.sum(w, axis=2)[..., None])
    return jnp.where(is_big[:, :, None], single, multiple)


def kernel(pts_co, params):
    pts_fea = pts_co
    co1, fea1 = _sa_level(params["sa1"], pts_co, pts_fea, 512, 0.2, 32)
    co2, fea2 = _sa_level(params["sa2"], co1, fea1, 128, 0.4, 64)

    gfeat = _global_feature(params["sa3"], fea2, 64)

    b, n2, _ = fea2.shape
    x = jnp.concatenate(
        [fea2, jnp.broadcast_to(gfeat[:, None, :], (b, n2, gfeat.shape[1]))],
        axis=2)
    fea2 = _mlp(x, params["fp3"])

    feat = _fp_interp(co1, co2, fea2)
    fea1 = _mlp(jnp.concatenate([fea1, feat], axis=2), params["fp2"])

    feat = _fp_interp(pts_co, co1, fea1)
    seg = _mlp(feat, params["fp1"])

    W, bvec = params["singleconv"]
    x = jnp.einsum('bnc,oc->bon', seg, W) + bvec[None, :, None]
    x = jax.nn.softmax(x, axis=2)
    return jnp.transpose(x, (0, 2, 1))


# speedup vs baseline: 2.9877x; 2.1312x over previous
import functools

import jax
import jax.numpy as jnp
from jax.experimental import pallas as pl
from jax.experimental.pallas import tpu as pltpu


def _fps_body(M, pts_ref, samp_ref, cox_ref, coy_ref, coz_ref):
    xs = pts_ref[0]
    ys = pts_ref[1]
    zs = pts_ref[2]
    b, n = xs.shape
    lane = jax.lax.broadcasted_iota(jnp.int32, (b, n), 1)
    lane_m = jax.lax.broadcasted_iota(jnp.int32, (b, M), 1)
    big_i = jnp.int32(n)

    def step(i, carry):
        min_d, last, samples, sx, sy, sz = carry
        oh = lane == last
        lx = jnp.max(jnp.where(oh, xs, -jnp.inf), axis=1, keepdims=True)
        ly = jnp.max(jnp.where(oh, ys, -jnp.inf), axis=1, keepdims=True)
        lz = jnp.max(jnp.where(oh, zs, -jnp.inf), axis=1, keepdims=True)
        sel = lane_m == i
        samples = jnp.where(sel, jnp.broadcast_to(last, (b, M)), samples)
        sx = jnp.where(sel, jnp.broadcast_to(lx, (b, M)), sx)
        sy = jnp.where(sel, jnp.broadcast_to(ly, (b, M)), sy)
        sz = jnp.where(sel, jnp.broadcast_to(lz, (b, M)), sz)
        dx = xs - lx
        dy = ys - ly
        dz = zs - lz
        d = dx * dx + dy * dy + dz * dz
        min_d = jnp.minimum(min_d, d)
        m = jnp.max(min_d, axis=1, keepdims=True)
        nxt = jnp.min(jnp.where(min_d == m, lane, big_i), axis=1,
                      keepdims=True)
        return min_d, nxt, samples, sx, sy, sz

    row_m = jax.lax.broadcasted_iota(jnp.int32, (b, M), 0)
    varying = lane_m + row_m
    init = (
        jnp.full((b, n), 1e10, dtype=jnp.float32),
        jnp.zeros((b, 1), dtype=jnp.int32),
        varying,
        varying.astype(jnp.float32),
        varying.astype(jnp.float32),
        varying.astype(jnp.float32),
    )
    _, _, samples, sx, sy, sz = jax.lax.fori_loop(0, M, step, init)
    samp_ref[...] = samples
    cox_ref[...] = sx
    coy_ref[...] = sy
    coz_ref[...] = sz


def _fps(pts_co, M):
    b, n, _ = pts_co.shape
    pts_t = jnp.transpose(pts_co, (2, 0, 1))
    samp, sx, sy, sz = pl.pallas_call(
        functools.partial(_fps_body, M),
        out_shape=[
            jax.ShapeDtypeStruct((b, M), jnp.int32),
            jax.ShapeDtypeStruct((b, M), jnp.float32),
            jax.ShapeDtypeStruct((b, M), jnp.float32),
            jax.ShapeDtypeStruct((b, M), jnp.float32),
        ],
    )(pts_t)
    new_co = jnp.stack([sx, sy, sz], axis=-1)
    return samp, new_co


def _topk_body(G, cb, emit_coords, px_ref, py_ref, pz_ref,
               cx_ref, cy_ref, cz_ref, *out_refs):
    px = px_ref[0]
    py = py_ref[0]
    pz = pz_ref[0]
    ib = pl.program_id(0)
    bb = cx_ref.shape[1]
    cb_ = cx_ref.shape[0]
    bmask = jax.lax.broadcasted_iota(jnp.int32, (cb_, bb), 1) == ib
    cx = jnp.sum(jnp.where(bmask, cx_ref[...], 0.0), axis=1, keepdims=True)
    cy = jnp.sum(jnp.where(bmask, cy_ref[...], 0.0), axis=1, keepdims=True)
    cz = jnp.sum(jnp.where(bmask, cz_ref[...], 0.0), axis=1, keepdims=True)
    n = px.shape[1]
    dx = cx - px
    dy = cy - py
    dz = cz - pz
    D = dx * dx + dy * dy + dz * dz
    lane = jax.lax.broadcasted_iota(jnp.int32, (cb, n), 1)
    lane_g = jax.lax.broadcasted_iota(jnp.int32, (cb, G), 1)
    row_g = jax.lax.broadcasted_iota(jnp.int32, (cb, G), 0)
    idx_acc = lane_g + row_g * 0
    d_acc = (lane_g + row_g).astype(jnp.float32)
    gx_acc = d_acc
    gy_acc = d_acc
    gz_acc = d_acc
    big = jnp.float32(jnp.inf)
    for k in range(G):
        m = jnp.min(D, axis=1, keepdims=True)
        sel = jnp.min(jnp.where(D == m, lane, n), axis=1, keepdims=True)
        oh = lane == sel
        slot = lane_g == k
        idx_acc = jnp.where(slot, sel, idx_acc)
        d_acc = jnp.where(slot, m, d_acc)
        if emit_coords:
            gx = jnp.max(jnp.where(oh, px, -big), axis=1, keepdims=True)
            gy = jnp.max(jnp.where(oh, py, -big), axis=1, keepdims=True)
            gz = jnp.max(jnp.where(oh, pz, -big), axis=1, keepdims=True)
            gx_acc = jnp.where(slot, gx, gx_acc)
            gy_acc = jnp.where(slot, gy, gy_acc)
            gz_acc = jnp.where(slot, gz, gz_acc)
        D = jnp.where(oh, big, D)
    out_refs[0][0] = idx_acc
    out_refs[1][0] = d_acc
    if emit_coords:
        out_refs[2][0] = gx_acc - cx
        out_refs[3][0] = gy_acc - cy
        out_refs[4][0] = gz_acc - cz
        out_refs[5][0] = gx_acc
        out_refs[6][0] = gy_acc
        out_refs[7][0] = gz_acc


def _topk(pts_co, new_co, G, emit_coords, cb=32):
    b, n, _ = pts_co.shape
    M = new_co.shape[1]
    px, py, pz = (pts_co[:, None, :, i] for i in range(3))
    cxt, cyt, czt = (jnp.transpose(new_co[:, :, i]) for i in range(3))
    grid = (b, M // cb)
    p_spec = pl.BlockSpec((1, 1, n), lambda ib, jm: (ib, 0, 0))
    c_spec = pl.BlockSpec((cb, b), lambda ib, jm: (jm, 0))
    o_spec = pl.BlockSpec((1, cb, G), lambda ib, jm: (ib, jm, 0))
    n_out = 8 if emit_coords else 2
    out_shape = ([jax.ShapeDtypeStruct((b, M, G), jnp.int32)]
                 + [jax.ShapeDtypeStruct((b, M, G), jnp.float32)] * (n_out - 1))
    outs = pl.pallas_call(
        functools.partial(_topk_body, G, cb, emit_coords),
        grid=grid,
        in_specs=[p_spec] * 3 + [c_spec] * 3,
        out_specs=[o_spec] * n_out,
        out_shape=out_shape,
    )(px, py, pz, cxt, cyt, czt)
    idx, dsq = outs[0], outs[1]
    if not emit_coords:
        return idx, dsq
    rel = jnp.stack(outs[2:5], axis=-1)
    gco = jnp.stack(outs[5:8], axis=-1)
    return idx, dsq, rel, gco


def _gather_nd(fea, idx):
    b = fea.shape[0]
    bi = jnp.arange(b).reshape((b,) + (1,) * (idx.ndim - 1))
    return fea[bi, idx]


def _bn_relu(x, g, beta):
    axes = tuple(range(x.ndim - 1))
    m = jnp.mean(x, axis=axes, keepdims=True)
    v = jnp.var(x, axis=axes, keepdims=True)
    return jax.nn.relu((x - m) / jnp.sqrt(v + 1e-5) * g + beta)


def _mlp(x, layers):
    for (W, bvec, g, beta) in layers:
        x = x @ W.T + bvec
        x = _bn_relu(x, g, beta)
    return x


def _nn3_interp(idx, dsq, fea_big):
    dd = jnp.sqrt(dsq + 1e-12)
    is_big = dd[:, :, 0] < 1e-5
    single = _gather_nd(fea_big, idx[:, :, 0])
    w = 1.0 / jnp.maximum(dd, 1e-8)
    gathered = _gather_nd(fea_big, idx)
    multiple = (jnp.sum(gathered * w[..., None], axis=2)
                / jnp.sum(w, axis=2)[..., None])
    return jnp.where(is_big[:, :, None], single, multiple)


def _interp_weights(idx, dsq, n_src):
    dd = jnp.sqrt(dsq + 1e-12)
    is_big = dd[:, :, :1] < 1e-5
    w = 1.0 / jnp.maximum(dd, 1e-8)
    wn = w / jnp.sum(w, axis=2, keepdims=True)
    w_eff = jnp.where(is_big, jnp.array([1.0, 0.0, 0.0], jnp.float32), wn)
    src = jnp.arange(n_src, dtype=jnp.int32)
    onehot = (idx[..., None] == src).astype(jnp.float32)
    return jnp.einsum('bmk,bmks->bms', w_eff, onehot)


def _global_feature(layers, pts_fea, n_sample):
    b, n, c = pts_fea.shape
    x = jnp.broadcast_to(pts_fea[:, None, :, :], (b, n_sample, n, c))
    x = _mlp(x, layers)
    return jnp.max(x, axis=2)[:, 0]


def kernel(pts_co, params):
    b = pts_co.shape[0]

    _, co1 = _fps(pts_co, 512)
    _, dsq1, rel1, gco1 = _topk(pts_co, co1, 32, True)
    within1 = (jnp.sqrt(dsq1 + 1e-12) <= 0.2)[..., None]
    rel1 = jnp.where(within1, rel1, rel1[:, :, :1, :])
    gco1 = jnp.where(within1, gco1, gco1[:, :, :1, :])
    x = jnp.concatenate([rel1, gco1], axis=3)
    fea1 = jnp.max(_mlp(x, params["sa1"]), axis=2)

    _, co2 = _fps(co1, 128)
    idx2, dsq2, rel2, _ = _topk(co1, co2, 64, True)
    w2 = (jnp.sqrt(dsq2 + 1e-12) <= 0.4)
    idx2 = jnp.where(w2, idx2, idx2[:, :, :1])
    rel2 = jnp.where(w2[..., None], rel2, rel2[:, :, :1, :])
    g_fea = _gather_nd(fea1, idx2)
    x = jnp.concatenate([rel2, g_fea], axis=3)
    x = _mlp(x, params["sa2"])
    fea2 = jnp.max(x, axis=2)

    gfeat = _global_feature(params["sa3"], fea2, 64)

    _, co3 = _fps(co2, 64)
    fea3 = jnp.broadcast_to(gfeat[:, None, :], (b, 64, gfeat.shape[1]))
    idxf3, dsqf3 = _topk(co3, co2, 3, False)
    feat3 = _nn3_interp(idxf3, dsqf3, fea3)
    fea2 = _mlp(jnp.concatenate([fea2, feat3], axis=2), params["fp3"])

    idxf2, dsqf2 = _topk(co2, co1, 3, False)
    feat = _nn3_interp(idxf2, dsqf2, fea2)
    fea1 = _mlp(jnp.concatenate([fea1, feat], axis=2), params["fp2"])

    idxf1, dsqf1 = _topk(co1, pts_co, 3, False)
    wm1 = _interp_weights(idxf1, dsqf1, 512)
    W1, b1, g1, be1 = params["fp1"][0]
    z = fea1 @ W1.T
    y1 = jnp.einsum('bns,bsc->bnc', wm1, z,
                    precision=jax.lax.Precision.HIGHEST) + b1
    x = _bn_relu(y1, g1, be1)
    seg = _mlp(x, params["fp1"][1:])

    W, bvec = params["singleconv"]
    x = jnp.einsum('bnc,oc->bon', seg, W) + bvec[None, :, None]
    x = jax.nn.softmax(x, axis=2)
    return jnp.transpose(x, (0, 2, 1))
